# Initial kernel scaffold; baseline (speedup 1.0000x reference)
import jax
import jax.numpy as jnp
from jax.experimental import pallas as pl


def kernel(node_feat, edge_index, params):
    raise NotImplementedError("write your pallas kernel here")


# speedup vs baseline: 4.9155x; 4.9155x over previous
import functools

import jax
import jax.numpy as jnp
from jax import lax
from jax.experimental import pallas as pl
from jax.experimental.pallas import tpu as pltpu
from jax.experimental.pallas import tpu_sc as plsc

_NC = 2
_NS = 16
_NW = _NC * _NS
_K = 128
_EPS_BN = 1e-5


@functools.lru_cache(maxsize=None)
def _make_agg(n, d, c):
    zr = ((n // _NS) + _K - 1) // _K * _K
    npad = _NS * zr
    ro = (n // _NS) // 8 * 8
    rem = n - ro * _NS

    mesh = plsc.VectorSubcoreMesh(core_axis_name="c", subcore_axis_name="s")

    def body(h_hbm, src_hbm, dst_hbm, out_hbm, sidx, didx, rows, acc, sem):
        cid = lax.axis_index("c")
        sid = lax.axis_index("s")
        w = cid * _NS + sid

        def zero_rows(i, carry):
            r = i // (d // 16)
            col = (i % (d // 16)) * 16
            rows[r, pl.ds(col, 16)] = jnp.zeros((16,), jnp.float32)
            return carry

        lax.fori_loop(0, _K * (d // 16), zero_rows, 0)
        for t in range(zr // _K):
            pltpu.sync_copy(rows, acc.at[pl.ds(sid * zr + t * _K, _K)])
        plsc.subcore_barrier()

        def step(j, carry):
            base = (w * c + j) * _K
            pltpu.sync_copy(src_hbm.at[pl.ds(base, _K)], sidx)
            pltpu.sync_copy(dst_hbm.at[pl.ds(base, _K)], didx)
            pltpu.async_copy(h_hbm.at[sidx], rows, sem).wait()
            pltpu.sync_copy(rows, acc.at[didx], add=True)
            return carry

        lax.fori_loop(0, c, step, 0)
        plsc.subcore_barrier()
        pltpu.sync_copy(acc.at[pl.ds(sid * ro, ro)],
                        out_hbm.at[cid, pl.ds(sid * ro, ro)])
        if rem:
            @pl.when(sid == _NS - 1)
            def _():
                pltpu.sync_copy(acc.at[pl.ds(ro * _NS, rem)],
                                out_hbm.at[cid, pl.ds(ro * _NS, rem)])

    return pl.kernel(
        body,
        out_type=jax.ShapeDtypeStruct((_NC, n, d), jnp.float32),
        mesh=mesh,
        compiler_params=pltpu.CompilerParams(use_tc_tiling_on_sc=False),
        scratch_types=[
            pltpu.VMEM((_K,), jnp.int32),
            pltpu.VMEM((_K,), jnp.int32),
            pltpu.VMEM((_K, d), jnp.float32),
            pltpu.VMEM_SHARED((npad, d), jnp.float32),
            pltpu.SemaphoreType.DMA,
        ],
    )


def _mlp_core(eps, h, a, W1, b1, g, be, W2, b2):
    rst = (1.0 + eps) * h + (a[0] + a[1])
    x = jnp.dot(rst, W1, preferred_element_type=jnp.float32) + b1
    mean = jnp.mean(x, axis=0, keepdims=True)
    xc = x - mean
    var = jnp.mean(xc * xc, axis=0, keepdims=True)
    x = xc * lax.rsqrt(var + _EPS_BN) * g + be
    x = jnp.maximum(x, 0.0)
    x = jnp.dot(x, W2, preferred_element_type=jnp.float32) + b2
    return jnp.maximum(x, 0.0)


def _mlp_body(eps_ref, h_ref, a_ref, W1_ref, b1_ref, g_ref, be_ref,
              W2_ref, b2_ref, o_ref):
    o_ref[...] = _mlp_core(eps_ref[0], h_ref[...], a_ref[...], W1_ref[...],
                           b1_ref[...], g_ref[...], be_ref[...], W2_ref[...],
                           b2_ref[...])


def _mlp_out_body(eps_ref, h_ref, a_ref, W1_ref, b1_ref, g_ref, be_ref,
                  W2_ref, b2_ref, Wo_ref, bo_ref, o_ref):
    hh = _mlp_core(eps_ref[0], h_ref[...], a_ref[...], W1_ref[...],
                   b1_ref[...], g_ref[...], be_ref[...], W2_ref[...],
                   b2_ref[...])
    o_ref[...] = jnp.dot(hh, Wo_ref[...],
                         preferred_element_type=jnp.float32) + bo_ref[...]


def _specs(n_vmem):
    return [pl.BlockSpec(memory_space=pltpu.SMEM)] + \
           [pl.BlockSpec(memory_space=pltpu.VMEM)] * n_vmem


def kernel(node_feat, edge_index, params):
    n = node_feat.shape[0]
    e = edge_index.shape[1]
    c = -(-e // (_NW * _K))
    epad = _NW * c * _K
    src = edge_index[0]
    dst = edge_index[1]
    if epad > e:
        src = jnp.concatenate([src, jnp.zeros((epad - e,), jnp.int32)])
        dst = jnp.concatenate([dst, jnp.full((epad - e,), n, jnp.int32)])
    h = node_feat
    layers = params['layers']
    for i, lp in enumerate(layers):
        d = h.shape[1]
        parts = _make_agg(n, d, c)(h, src, dst)
        hd = lp['W1'].shape[1]
        args = (lp['eps'].reshape(1), h, parts, lp['W1'],
                lp['b1'].reshape(1, -1), lp['gamma'].reshape(1, -1),
                lp['beta'].reshape(1, -1), lp['W2'], lp['b2'].reshape(1, -1))
        if i + 1 < len(layers):
            h = pl.pallas_call(
                _mlp_body,
                out_shape=jax.ShapeDtypeStruct((n, hd), jnp.float32),
                in_specs=_specs(8),
            )(*args)
        else:
            out_d = params['Wo'].shape[1]
            h = pl.pallas_call(
                _mlp_out_body,
                out_shape=jax.ShapeDtypeStruct((n, out_d), jnp.float32),
                in_specs=_specs(10),
            )(*args, params['Wo'], params['bo'].reshape(1, -1))
    return h
